# baseline (speedup 1.0000x reference)
import jax
import jax.numpy as jnp
from jax.experimental import pallas as pl

_N, _D, _G = 10000, 256, 64
_R = 5000
_C = 128


def _fm_kernel(x_ref, mask_ref, m_ref, xm_ref):
    i = pl.program_id(0)
    j = pl.program_id(1)
    s = jax.nn.sigmoid(mask_ref[...])
    xm_ref[...] = x_ref[...] * s

    @pl.when((i == 0) & (j == 0))
    def _():
        m_ref[...] = jnp.zeros_like(m_ref)


def kernel(x, edge_index, batch, train_mask):
    mask2 = train_mask.reshape(1, _D)
    m, xm = pl.pallas_call(
        _fm_kernel,
        grid=(_N // _R, _D // _C),
        in_specs=[
            pl.BlockSpec((_R, _C), lambda i, j: (i, j)),
            pl.BlockSpec((1, _C), lambda i, j: (0, j)),
        ],
        out_specs=[
            pl.BlockSpec((_G, _D), lambda i, j: (0, 0)),
            pl.BlockSpec((_R, _C), lambda i, j: (i, j)),
        ],
        out_shape=[
            jax.ShapeDtypeStruct((_G, _D), jnp.float32),
            jax.ShapeDtypeStruct((_N, _D), jnp.float32),
        ],
    )(x, mask2)
    return m, xm


# speedup vs baseline: 1.0018x; 1.0018x over previous
import jax
import jax.numpy as jnp
from jax.experimental import pallas as pl

_N, _D, _G = 10000, 256, 64
_R = 5000


def _fm_kernel(x_ref, b_ref, mask_ref, m_ref, xm_ref):
    i = pl.program_id(0)
    s = jax.nn.sigmoid(mask_ref[...])
    xm = x_ref[...] * s
    xm_ref[...] = xm
    b = b_ref[0]
    gids = jax.lax.broadcasted_iota(jnp.int32, (_G, _R), 0)
    onehot_t = (gids == b).astype(jnp.bfloat16)
    part = jax.lax.dot_general(
        onehot_t, xm.astype(jnp.bfloat16), (((1,), (0,)), ((), ())),
        preferred_element_type=jnp.float32)

    @pl.when(i == 0)
    def _():
        m_ref[...] = jnp.zeros_like(m_ref)

    m_ref[...] += part


def kernel(x, edge_index, batch, train_mask):
    b2 = batch.reshape(_N // _R, 1, _R)
    mask2 = train_mask.reshape(1, _D)
    m, xm = pl.pallas_call(
        _fm_kernel,
        grid=(_N // _R,),
        in_specs=[
            pl.BlockSpec((_R, _D), lambda i: (i, 0)),
            pl.BlockSpec((1, 1, _R), lambda i: (i, 0, 0)),
            pl.BlockSpec((1, _D), lambda i: (0, 0)),
        ],
        out_specs=[
            pl.BlockSpec((_G, _D), lambda i: (0, 0)),
            pl.BlockSpec((_R, _D), lambda i: (i, 0)),
        ],
        out_shape=[
            jax.ShapeDtypeStruct((_G, _D), jnp.float32),
            jax.ShapeDtypeStruct((_N, _D), jnp.float32),
        ],
    )(x, b2, mask2)
    return m, xm
